# baseline (speedup 1.0000x reference)
import functools

import jax
import jax.numpy as jnp
from jax import lax
from jax.experimental import pallas as pl
from jax.experimental.pallas import tpu as pltpu
from jax.experimental.pallas import tpu_sc as plsc

_SC_NUM_CORES = 2
_SC_NUM_SUBCORES = 16
_SC_NUM_WORKERS = _SC_NUM_CORES * _SC_NUM_SUBCORES

_BN = 512


def _sc_gather(table, idx):
    B = idx.shape[0]
    V, E = table.shape
    assert B % (8 * _SC_NUM_WORKERS) == 0
    b_per_w = B // _SC_NUM_WORKERS

    mesh = plsc.VectorSubcoreMesh(core_axis_name="c", subcore_axis_name="s")

    @functools.partial(
        pl.kernel,
        mesh=mesh,
        out_type=jax.ShapeDtypeStruct((B, E), jnp.float32),
        scratch_types=[
            pltpu.VMEM((b_per_w,), jnp.int32),
            pltpu.VMEM((b_per_w, E), jnp.float32),
            pltpu.SemaphoreType.DMA,
        ],
        compiler_params=pltpu.CompilerParams(use_tc_tiling_on_sc=False),
    )
    def gather_kernel(table_hbm, idx_hbm, out_hbm, idx_v, rows_v, sem):
        wid = lax.axis_index("s") * _SC_NUM_CORES + lax.axis_index("c")
        base = wid * b_per_w
        pltpu.sync_copy(idx_hbm.at[pl.ds(base, b_per_w)], idx_v)
        pltpu.async_copy(table_hbm.at[idx_v], rows_v, sem).wait()
        pltpu.sync_copy(rows_v, out_hbm.at[pl.ds(base, b_per_w)])

    return gather_kernel(table, idx)


def _lse_body(emb_ref, w_ref, b_ref, lse_ref, m_ref, s_ref, *, bn, v, nv):
    iv = pl.program_id(0)
    logits = lax.dot_general(
        emb_ref[...], w_ref[...],
        (((1,), (1,)), ((), ())),
        preferred_element_type=jnp.float32,
    ) + b_ref[...]
    col = iv * bn + lax.broadcasted_iota(jnp.int32, (1, bn), 1)
    logits = jnp.where(col < v, logits, -jnp.inf)
    tile_max = jnp.max(logits, axis=1, keepdims=True)
    m_prev = jnp.where(iv == 0, -jnp.inf, m_ref[...])
    s_prev = jnp.where(iv == 0, 0.0, s_ref[...])
    m_new = jnp.maximum(m_prev, tile_max)
    s_new = s_prev * jnp.exp(m_prev - m_new) + jnp.sum(
        jnp.exp(logits - m_new), axis=1, keepdims=True)
    m_ref[...] = m_new
    s_ref[...] = s_new

    @pl.when(iv == nv - 1)
    def _():
        lse_ref[...] = m_new + jnp.log(s_new)


def _project_body(emb_ref, w_ref, b_ref, lse_ref, out_ref):
    logits = lax.dot_general(
        emb_ref[...], w_ref[...],
        (((1,), (1,)), ((), ())),
        preferred_element_type=jnp.float32,
    ) + b_ref[...]
    out_ref[...] = logits - lse_ref[...]


def kernel(context_word, emb_table, W, b):
    B = context_word.shape[0]
    V, E = emb_table.shape
    bn = _BN
    nv = pl.cdiv(V, bn)

    emb = _sc_gather(emb_table, context_word)
    b2 = b.reshape(1, V)

    lse = pl.pallas_call(
        functools.partial(_lse_body, bn=bn, v=V, nv=nv),
        grid=(nv,),
        in_specs=[
            pl.BlockSpec((B, E), lambda iv: (0, 0)),
            pl.BlockSpec((bn, E), lambda iv: (iv, 0)),
            pl.BlockSpec((1, bn), lambda iv: (0, iv)),
        ],
        out_specs=pl.BlockSpec((B, 1), lambda iv: (0, 0)),
        out_shape=jax.ShapeDtypeStruct((B, 1), jnp.float32),
        scratch_shapes=[
            pltpu.VMEM((B, 1), jnp.float32),
            pltpu.VMEM((B, 1), jnp.float32),
        ],
        compiler_params=pltpu.CompilerParams(
            dimension_semantics=("arbitrary",),
        ),
    )(emb, W, b2)

    out = pl.pallas_call(
        _project_body,
        grid=(nv,),
        in_specs=[
            pl.BlockSpec((B, E), lambda iv: (0, 0)),
            pl.BlockSpec((bn, E), lambda iv: (iv, 0)),
            pl.BlockSpec((1, bn), lambda iv: (0, iv)),
            pl.BlockSpec((B, 1), lambda iv: (0, 0)),
        ],
        out_specs=pl.BlockSpec((B, bn), lambda iv: (0, iv)),
        out_shape=jax.ShapeDtypeStruct((B, V), jnp.float32),
        compiler_params=pltpu.CompilerParams(
            dimension_semantics=("parallel",),
        ),
    )(emb, W, b2, lse)
    return out


# speedup vs baseline: 1.0066x; 1.0066x over previous
import functools

import jax
import jax.numpy as jnp
from jax import lax
from jax.experimental import pallas as pl
from jax.experimental.pallas import tpu as pltpu
from jax.experimental.pallas import tpu_sc as plsc

_SC_NUM_CORES = 2
_SC_NUM_SUBCORES = 16
_SC_NUM_WORKERS = _SC_NUM_CORES * _SC_NUM_SUBCORES

_BN = 512


def _sc_gather(table, idx):
    B = idx.shape[0]
    V, E = table.shape
    assert B % (8 * _SC_NUM_WORKERS) == 0
    b_per_w = B // _SC_NUM_WORKERS

    mesh = plsc.VectorSubcoreMesh(core_axis_name="c", subcore_axis_name="s")

    @functools.partial(
        pl.kernel,
        mesh=mesh,
        out_type=jax.ShapeDtypeStruct((B, E), jnp.float32),
        scratch_types=[
            pltpu.VMEM((b_per_w,), jnp.int32),
            pltpu.VMEM((b_per_w, E), jnp.float32),
            pltpu.SemaphoreType.DMA,
        ],
        compiler_params=pltpu.CompilerParams(use_tc_tiling_on_sc=False),
    )
    def gather_kernel(table_hbm, idx_hbm, out_hbm, idx_v, rows_v, sem):
        wid = lax.axis_index("s") * _SC_NUM_CORES + lax.axis_index("c")
        base = wid * b_per_w
        pltpu.sync_copy(idx_hbm.at[pl.ds(base, b_per_w)], idx_v)
        pltpu.async_copy(table_hbm.at[idx_v], rows_v, sem).wait()
        pltpu.sync_copy(rows_v, out_hbm.at[pl.ds(base, b_per_w)])

    return gather_kernel(table, idx)


def _lse_body(emb_ref, w_ref, b_ref, lse_ref, m_ref, s_ref, *, bn, v, nv):
    iv = pl.program_id(0)
    logits = lax.dot_general(
        emb_ref[...].astype(jnp.bfloat16), w_ref[...].astype(jnp.bfloat16),
        (((1,), (1,)), ((), ())),
        preferred_element_type=jnp.float32,
    ) + b_ref[...]
    col = iv * bn + lax.broadcasted_iota(jnp.int32, (1, bn), 1)
    logits = jnp.where(col < v, logits, -jnp.inf)
    tile_max = jnp.max(logits, axis=1, keepdims=True)
    m_prev = jnp.where(iv == 0, -jnp.inf, m_ref[...])
    s_prev = jnp.where(iv == 0, 0.0, s_ref[...])
    m_new = jnp.maximum(m_prev, tile_max)
    s_new = s_prev * jnp.exp(m_prev - m_new) + jnp.sum(
        jnp.exp(logits - m_new), axis=1, keepdims=True)
    m_ref[...] = m_new
    s_ref[...] = s_new

    @pl.when(iv == nv - 1)
    def _():
        lse_ref[...] = m_new + jnp.log(s_new)


def _project_body(emb_ref, w_ref, b_ref, lse_ref, out_ref):
    logits = lax.dot_general(
        emb_ref[...].astype(jnp.bfloat16), w_ref[...].astype(jnp.bfloat16),
        (((1,), (1,)), ((), ())),
        preferred_element_type=jnp.float32,
    ) + b_ref[...]
    out_ref[...] = logits - lse_ref[...]


def kernel(context_word, emb_table, W, b):
    B = context_word.shape[0]
    V, E = emb_table.shape
    bn = _BN
    nv = pl.cdiv(V, bn)

    emb = _sc_gather(emb_table, context_word)
    b2 = b.reshape(1, V)

    lse = pl.pallas_call(
        functools.partial(_lse_body, bn=bn, v=V, nv=nv),
        grid=(nv,),
        in_specs=[
            pl.BlockSpec((B, E), lambda iv: (0, 0)),
            pl.BlockSpec((bn, E), lambda iv: (iv, 0)),
            pl.BlockSpec((1, bn), lambda iv: (0, iv)),
        ],
        out_specs=pl.BlockSpec((B, 1), lambda iv: (0, 0)),
        out_shape=jax.ShapeDtypeStruct((B, 1), jnp.float32),
        scratch_shapes=[
            pltpu.VMEM((B, 1), jnp.float32),
            pltpu.VMEM((B, 1), jnp.float32),
        ],
        compiler_params=pltpu.CompilerParams(
            dimension_semantics=("arbitrary",),
        ),
    )(emb, W, b2)

    out = pl.pallas_call(
        _project_body,
        grid=(nv,),
        in_specs=[
            pl.BlockSpec((B, E), lambda iv: (0, 0)),
            pl.BlockSpec((bn, E), lambda iv: (iv, 0)),
            pl.BlockSpec((1, bn), lambda iv: (0, iv)),
            pl.BlockSpec((B, 1), lambda iv: (0, 0)),
        ],
        out_specs=pl.BlockSpec((B, bn), lambda iv: (0, iv)),
        out_shape=jax.ShapeDtypeStruct((B, V), jnp.float32),
        compiler_params=pltpu.CompilerParams(
            dimension_semantics=("parallel",),
        ),
    )(emb, W, b2, lse)
    return out


# speedup vs baseline: 1.4219x; 1.4126x over previous
import functools

import jax
import jax.numpy as jnp
from jax import lax
from jax.experimental import pallas as pl
from jax.experimental.pallas import tpu as pltpu
from jax.experimental.pallas import tpu_sc as plsc

_SC_NUM_CORES = 2
_SC_NUM_SUBCORES = 16
_SC_NUM_WORKERS = _SC_NUM_CORES * _SC_NUM_SUBCORES

_BN = 512
_LANES = 128


def _sc_gather(table, idx):
    B = idx.shape[0]
    V, E = table.shape
    assert B % (8 * _SC_NUM_WORKERS) == 0
    b_per_w = B // _SC_NUM_WORKERS

    mesh = plsc.VectorSubcoreMesh(core_axis_name="c", subcore_axis_name="s")

    @functools.partial(
        pl.kernel,
        mesh=mesh,
        out_type=jax.ShapeDtypeStruct((B, E), jnp.float32),
        scratch_types=[
            pltpu.VMEM((b_per_w,), jnp.int32),
            pltpu.VMEM((b_per_w, E), jnp.float32),
            pltpu.SemaphoreType.DMA,
        ],
        compiler_params=pltpu.CompilerParams(use_tc_tiling_on_sc=False),
    )
    def gather_kernel(table_hbm, idx_hbm, out_hbm, idx_v, rows_v, sem):
        wid = lax.axis_index("s") * _SC_NUM_CORES + lax.axis_index("c")
        base = wid * b_per_w
        pltpu.sync_copy(idx_hbm.at[pl.ds(base, b_per_w)], idx_v)
        pltpu.async_copy(table_hbm.at[idx_v], rows_v, sem).wait()
        pltpu.sync_copy(rows_v, out_hbm.at[pl.ds(base, b_per_w)])

    return gather_kernel(table, idx)


def _dot_nt(a, bm):
    return lax.dot_general(
        a, bm, (((1,), (1,)), ((), ())), preferred_element_type=jnp.float32)


def _lse_body(emb_ref, w_ref, b_ref, lse_ref, m_ref, s_ref, *, bn, nv):
    iv = pl.program_id(0)
    x = _dot_nt(emb_ref[...], w_ref[...]) + b_ref[...]
    g = bn // _LANES
    xs = [lax.slice_in_dim(x, k * _LANES, (k + 1) * _LANES, axis=1)
          for k in range(g)]
    cm = xs[0]
    for k in range(1, g):
        cm = jnp.maximum(cm, xs[k])
    m_prev = jnp.where(iv == 0, -jnp.inf, m_ref[...])
    s_prev = jnp.where(iv == 0, 0.0, s_ref[...])
    m_new = jnp.maximum(m_prev, cm)
    ssum = jnp.exp(xs[0] - m_new)
    for k in range(1, g):
        ssum = ssum + jnp.exp(xs[k] - m_new)
    s_new = s_prev * jnp.exp(m_prev - m_new) + ssum
    m_ref[...] = m_new
    s_ref[...] = s_new

    @pl.when(iv == nv - 1)
    def _():
        mtot = jnp.max(m_new, axis=1, keepdims=True)
        stot = jnp.sum(s_new * jnp.exp(m_new - mtot), axis=1, keepdims=True)
        lse = mtot + jnp.log(stot)
        lse_ref[...] = jnp.broadcast_to(lse, lse_ref.shape)


def _project_body(emb_ref, w_ref, b_ref, lse_ref, out_ref, *, bn):
    x = _dot_nt(emb_ref[...], w_ref[...]) + b_ref[...]
    lse = lse_ref[...]
    for k in range(bn // _LANES):
        xk = lax.slice_in_dim(x, k * _LANES, (k + 1) * _LANES, axis=1)
        out_ref[:, pl.ds(k * _LANES, _LANES)] = xk - lse


def kernel(context_word, emb_table, W, b):
    B = context_word.shape[0]
    V, E = emb_table.shape
    bn = _BN
    nv = pl.cdiv(V, bn)
    vpad = nv * bn

    emb = _sc_gather(emb_table, context_word).astype(jnp.bfloat16)
    wp = jnp.pad(W, ((0, vpad - V), (0, 0))).astype(jnp.bfloat16)
    bp = jnp.pad(b.reshape(1, V), ((0, 0), (0, vpad - V)),
                 constant_values=-jnp.inf)

    lse = pl.pallas_call(
        functools.partial(_lse_body, bn=bn, nv=nv),
        grid=(nv,),
        in_specs=[
            pl.BlockSpec((B, E), lambda iv: (0, 0)),
            pl.BlockSpec((bn, E), lambda iv: (iv, 0)),
            pl.BlockSpec((1, bn), lambda iv: (0, iv)),
        ],
        out_specs=pl.BlockSpec((B, _LANES), lambda iv: (0, 0)),
        out_shape=jax.ShapeDtypeStruct((B, _LANES), jnp.float32),
        scratch_shapes=[
            pltpu.VMEM((B, _LANES), jnp.float32),
            pltpu.VMEM((B, _LANES), jnp.float32),
        ],
        compiler_params=pltpu.CompilerParams(
            dimension_semantics=("arbitrary",),
        ),
    )(emb, wp, bp)

    out = pl.pallas_call(
        functools.partial(_project_body, bn=bn),
        grid=(nv,),
        in_specs=[
            pl.BlockSpec((B, E), lambda iv: (0, 0)),
            pl.BlockSpec((bn, E), lambda iv: (iv, 0)),
            pl.BlockSpec((1, bn), lambda iv: (0, iv)),
            pl.BlockSpec((B, _LANES), lambda iv: (0, 0)),
        ],
        out_specs=pl.BlockSpec((B, bn), lambda iv: (0, iv)),
        out_shape=jax.ShapeDtypeStruct((B, V), jnp.float32),
        compiler_params=pltpu.CompilerParams(
            dimension_semantics=("parallel",),
        ),
    )(emb, wp, bp, lse)
    return out
